# Initial kernel scaffold; baseline (speedup 1.0000x reference)
import jax
import jax.numpy as jnp
from jax.experimental import pallas as pl


def kernel(input, target, table):
    raise NotImplementedError("write your pallas kernel here")


# speedup vs baseline: 1.2478x; 1.2478x over previous
import functools

import jax
import jax.numpy as jnp
from jax import lax
from jax.experimental import pallas as pl
from jax.experimental.pallas import tpu as pltpu
from jax.experimental.pallas import tpu_sc as plsc

VOCAB = 1000
NB = 1024
NS_SEQ = 200
NTOK = NB * NS_SEQ

NC = 2
NSUB = 16
NW = NC * NSUB
PER_W = NTOK // NW
CK = 64
NCH = PER_W // CK
LANES = 16


def _lse_tc(table):
    def body(t_ref, o_ref):
        x = t_ref[...]
        m = jnp.max(x, axis=1, keepdims=True)
        s = jnp.sum(jnp.exp(x - m), axis=1, keepdims=True)
        o_ref[...] = jnp.log(s) + m

    return pl.pallas_call(
        body,
        out_shape=jax.ShapeDtypeStruct((VOCAB, 1), jnp.float32),
    )(table)


@functools.partial(
    pl.kernel,
    out_type=[
        jax.ShapeDtypeStruct((NTOK, VOCAB), jnp.float32),
        jax.ShapeDtypeStruct((NW, LANES), jnp.float32),
    ],
    mesh=plsc.VectorSubcoreMesh(core_axis_name="c", subcore_axis_name="s"),
    compiler_params=pltpu.CompilerParams(
        needs_layout_passes=False, use_tc_tiling_on_sc=False
    ),
    scratch_types=[
        pltpu.VMEM((CK,), jnp.int32),
        pltpu.VMEM((CK,), jnp.int32),
        pltpu.VMEM((CK, VOCAB), jnp.float32),
        pltpu.VMEM((VOCAB,), jnp.float32),
        pltpu.VMEM((LANES,), jnp.float32),
        pltpu.SemaphoreType.DMA,
    ],
)
def _sc_gather(table_hbm, idx_hbm, tgt_hbm, lse_hbm,
               out_hbm, loss_hbm,
               idx_v, tgt_v, rows_v, lse_v, part_v, sem):
    wid = lax.axis_index("s") * NC + lax.axis_index("c")
    base0 = wid * PER_W
    pltpu.sync_copy(lse_hbm, lse_v)

    def body(c, acc):
        base = base0 + c * CK
        pltpu.sync_copy(idx_hbm.at[pl.ds(base, CK)], idx_v)
        pltpu.sync_copy(tgt_hbm.at[pl.ds(base, CK)], tgt_v)
        pltpu.async_copy(table_hbm.at[idx_v], rows_v, sem).wait()
        pltpu.sync_copy(rows_v, out_hbm.at[pl.ds(base, CK)])
        for j in range(CK // LANES):
            inp16 = idx_v[pl.ds(j * LANES, LANES)]
            tgt16 = tgt_v[pl.ds(j * LANES, LANES)]
            lse16 = plsc.load_gather(lse_v, [inp16])
            rid = lax.iota(jnp.int32, LANES) + (j * LANES)
            tv = plsc.load_gather(rows_v, [rid, tgt16])
            acc = acc + (lse16 - tv)
        return acc

    acc = lax.fori_loop(0, NCH, body, jnp.zeros((LANES,), jnp.float32))
    part_v[...] = acc
    pltpu.sync_copy(part_v, loss_hbm.at[wid])


def kernel(input, target, table):
    lse = _lse_tc(table).reshape(VOCAB)
    idx = input.reshape(NTOK)
    tgt = target.reshape(NTOK)
    out, parts = _sc_gather(table, idx, tgt, lse)
    logits = out.reshape(NB, NS_SEQ, VOCAB)
    loss = jnp.sum(parts) / jnp.float32(NTOK)
    return (logits, loss)


# speedup vs baseline: 1.3323x; 1.0677x over previous
import functools

import jax
import jax.numpy as jnp
from jax import lax
from jax.experimental import pallas as pl
from jax.experimental.pallas import tpu as pltpu
from jax.experimental.pallas import tpu_sc as plsc

VOCAB = 1000
NB = 1024
NS_SEQ = 200
NTOK = NB * NS_SEQ

NC = 2
NSUB = 16
NW = NC * NSUB
PER_W = NTOK // NW
CK = 32
NCH = PER_W // CK
LANES = 16


def _lse_tc(table):
    def body(t_ref, o_ref):
        x = t_ref[...]
        m = jnp.max(x, axis=1, keepdims=True)
        s = jnp.sum(jnp.exp(x - m), axis=1, keepdims=True)
        o_ref[...] = jnp.log(s) + m

    return pl.pallas_call(
        body,
        out_shape=jax.ShapeDtypeStruct((VOCAB, 1), jnp.float32),
    )(table)


@functools.partial(
    pl.kernel,
    out_type=[
        jax.ShapeDtypeStruct((NTOK, VOCAB), jnp.float32),
        jax.ShapeDtypeStruct((NW, LANES), jnp.float32),
    ],
    mesh=plsc.VectorSubcoreMesh(core_axis_name="c", subcore_axis_name="s"),
    compiler_params=pltpu.CompilerParams(
        needs_layout_passes=False, use_tc_tiling_on_sc=False
    ),
    scratch_types=[
        pltpu.VMEM((PER_W,), jnp.int32),
        pltpu.VMEM((PER_W,), jnp.int32),
        pltpu.VMEM((CK, VOCAB), jnp.float32),
        pltpu.VMEM((CK, VOCAB), jnp.float32),
        pltpu.VMEM((VOCAB,), jnp.float32),
        pltpu.VMEM((LANES,), jnp.float32),
        pltpu.SemaphoreType.DMA,
        pltpu.SemaphoreType.DMA,
        pltpu.SemaphoreType.DMA,
        pltpu.SemaphoreType.DMA,
    ],
)
def _sc_gather(table_hbm, idx_hbm, tgt_hbm, lse_hbm,
               out_hbm, loss_hbm,
               idx_all, tgt_all, rows0, rows1, lse_v, part_v,
               gsem0, gsem1, ssem0, ssem1):
    wid = lax.axis_index("s") * NC + lax.axis_index("c")
    base0 = wid * PER_W
    rows = (rows0, rows1)
    gsem = (gsem0, gsem1)
    ssem = (ssem0, ssem1)

    pltpu.sync_copy(lse_hbm, lse_v)
    pltpu.sync_copy(idx_hbm.at[wid], idx_all)
    pltpu.sync_copy(tgt_hbm.at[wid], tgt_all)

    def gather_start(c, b):
        pltpu.async_copy(
            table_hbm.at[idx_all.at[pl.ds(c * CK, CK)]], rows[b], gsem[b])

    def gather_wait(c, b):
        pltpu.make_async_copy(
            table_hbm.at[idx_all.at[pl.ds(c * CK, CK)]], rows[b], gsem[b]
        ).wait()

    def scatter_start(c, b):
        pltpu.async_copy(
            rows[b], out_hbm.at[pl.ds(base0 + c * CK, CK)], ssem[b])

    def scatter_wait(c, b):
        pltpu.make_async_copy(
            rows[b], out_hbm.at[pl.ds(base0 + c * CK, CK)], ssem[b]
        ).wait()

    def compute(c, b, acc):
        for j in range(CK // LANES):
            off = c * CK + j * LANES
            inp16 = idx_all[pl.ds(off, LANES)]
            tgt16 = tgt_all[pl.ds(off, LANES)]
            lse16 = plsc.load_gather(lse_v, [inp16])
            rid = lax.iota(jnp.int32, LANES) + (j * LANES)
            tv = plsc.load_gather(rows[b], [rid, tgt16])
            acc = acc + (lse16 - tv)
        return acc

    def slot(c, b, acc, first=False, last=False):
        if not first:
            scatter_wait(c - 1, 1 - b)
        if not last:
            gather_start(c + 1, 1 - b)
        gather_wait(c, b)
        scatter_start(c, b)
        return compute(c, b, acc)

    acc = jnp.zeros((LANES,), jnp.float32)
    gather_start(0, 0)
    acc = slot(0, 0, acc, first=True)

    def body(i, acc):
        acc = slot(2 * i + 1, 1, acc)
        acc = slot(2 * i + 2, 0, acc)
        return acc

    acc = lax.fori_loop(0, (NCH - 2) // 2, body, acc)
    acc = slot(NCH - 1, 1, acc, last=True)
    scatter_wait(NCH - 1, 1)

    part_v[...] = acc
    pltpu.sync_copy(part_v, loss_hbm.at[wid])


def kernel(input, target, table):
    lse = _lse_tc(table).reshape(VOCAB)
    idx = input.reshape(NW, PER_W)
    tgt = target.reshape(NW, PER_W)
    out, parts = _sc_gather(table, idx, tgt, lse)
    logits = out.reshape(NB, NS_SEQ, VOCAB)
    loss = jnp.sum(parts) / jnp.float32(NTOK)
    return (logits, loss)


# speedup vs baseline: 1.3329x; 1.0005x over previous
import functools

import jax
import jax.numpy as jnp
from jax import lax
from jax.experimental import pallas as pl
from jax.experimental.pallas import tpu as pltpu
from jax.experimental.pallas import tpu_sc as plsc

VOCAB = 1000
NB = 1024
NS_SEQ = 200
NTOK = NB * NS_SEQ

NC = 2
NSUB = 16
NW = NC * NSUB
PER_W = NTOK // NW
NB_W = PER_W // NS_SEQ
CK = 40
CPB = NS_SEQ // CK
NCH = PER_W // CK
LANES = 16


def _lse_tc(table):
    def body(t_ref, o_ref):
        x = t_ref[...]
        m = jnp.max(x, axis=1, keepdims=True)
        s = jnp.sum(jnp.exp(x - m), axis=1, keepdims=True)
        o_ref[...] = jnp.log(s) + m

    return pl.pallas_call(
        body,
        out_shape=jax.ShapeDtypeStruct((VOCAB, 1), jnp.float32),
    )(table)


@functools.partial(
    pl.kernel,
    out_type=[
        jax.ShapeDtypeStruct((NB, NS_SEQ, VOCAB), jnp.float32),
        jax.ShapeDtypeStruct((NW, LANES), jnp.float32),
    ],
    mesh=plsc.VectorSubcoreMesh(core_axis_name="c", subcore_axis_name="s"),
    compiler_params=pltpu.CompilerParams(
        needs_layout_passes=False, use_tc_tiling_on_sc=False
    ),
    scratch_types=[
        pltpu.VMEM((PER_W + LANES,), jnp.int32),
        pltpu.VMEM((PER_W + LANES,), jnp.int32),
        pltpu.VMEM((CK, VOCAB), jnp.float32),
        pltpu.VMEM((CK, VOCAB), jnp.float32),
        pltpu.VMEM((VOCAB,), jnp.float32),
        pltpu.VMEM((LANES,), jnp.float32),
        pltpu.SemaphoreType.DMA,
        pltpu.SemaphoreType.DMA,
        pltpu.SemaphoreType.DMA,
        pltpu.SemaphoreType.DMA,
    ],
)
def _sc_gather(table_hbm, idx_hbm, tgt_hbm, lse_hbm,
               out_hbm, loss_hbm,
               idx_all, tgt_all, rows0, rows1, lse_v, part_v,
               gsem0, gsem1, ssem0, ssem1):
    wid = lax.axis_index("s") * NC + lax.axis_index("c")
    rows = (rows0, rows1)
    gsem = (gsem0, gsem1)
    ssem = (ssem0, ssem1)

    pltpu.sync_copy(lse_hbm, lse_v)
    pltpu.sync_copy(idx_hbm.at[wid], idx_all.at[pl.ds(0, PER_W)])
    pltpu.sync_copy(tgt_hbm.at[wid], tgt_all.at[pl.ds(0, PER_W)])

    def out_slice(c):
        batch = wid * NB_W + c // CPB
        s0 = (c % CPB) * CK
        return out_hbm.at[batch, pl.ds(s0, CK)]

    def gather_start(c, b):
        pltpu.async_copy(
            table_hbm.at[idx_all.at[pl.ds(c * CK, CK)]], rows[b], gsem[b])

    def gather_wait(c, b):
        pltpu.make_async_copy(
            table_hbm.at[idx_all.at[pl.ds(c * CK, CK)]], rows[b], gsem[b]
        ).wait()

    def scatter_start(c, b):
        pltpu.async_copy(rows[b], out_slice(c), ssem[b])

    def scatter_wait(c, b):
        pltpu.make_async_copy(rows[b], out_slice(c), ssem[b]).wait()

    iota = lax.iota(jnp.int32, LANES)
    tail_mask = iota < (CK % LANES)

    def compute(c, b, acc):
        base = c * CK
        for j in range(-(-CK // LANES)):
            off = base + j * LANES
            inp16 = idx_all[pl.ds(off, LANES)]
            tgt16 = tgt_all[pl.ds(off, LANES)]
            rid = iota + (j * LANES)
            if (j + 1) * LANES <= CK:
                lse16 = plsc.load_gather(lse_v, [inp16])
                tv = plsc.load_gather(rows[b], [rid, tgt16])
                acc = acc + (lse16 - tv)
            else:
                lse16 = plsc.load_gather(lse_v, [inp16], mask=tail_mask)
                tv = plsc.load_gather(rows[b], [rid, tgt16], mask=tail_mask)
                acc = acc + jnp.where(tail_mask, lse16 - tv, 0.0)
        return acc

    def slot(c, b, acc, first=False, last=False):
        if not first:
            scatter_wait(c - 1, 1 - b)
        if not last:
            gather_start(c + 1, 1 - b)
        gather_wait(c, b)
        scatter_start(c, b)
        return compute(c, b, acc)

    acc = jnp.zeros((LANES,), jnp.float32)
    gather_start(0, 0)
    acc = slot(0, 0, acc, first=True)

    def body(i, acc):
        acc = slot(2 * i + 1, 1, acc)
        acc = slot(2 * i + 2, 0, acc)
        return acc

    acc = lax.fori_loop(0, (NCH - 2) // 2, body, acc)
    acc = slot(NCH - 1, 1, acc, last=True)
    scatter_wait(NCH - 1, 1)

    part_v[...] = acc
    pltpu.sync_copy(part_v, loss_hbm.at[wid])


def kernel(input, target, table):
    lse = _lse_tc(table).reshape(VOCAB)
    idx = input.reshape(NW, PER_W)
    tgt = target.reshape(NW, PER_W)
    logits, parts = _sc_gather(table, idx, tgt, lse)
    loss = jnp.sum(parts) / jnp.float32(NTOK)
    return (logits, loss)
